# Initial kernel scaffold; baseline (speedup 1.0000x reference)
import jax
import jax.numpy as jnp
from jax.experimental import pallas as pl


def kernel(h, edge_index, W_self, W_neigh):
    raise NotImplementedError("write your pallas kernel here")


# speedup vs baseline: 2.4030x; 2.4030x over previous
import jax
import jax.numpy as jnp
from jax import lax
from jax.experimental import pallas as pl
from jax.experimental.pallas import tpu as pltpu
from jax.experimental.pallas import tpu_sc as plsc

N = 10000
E = 320000
D = 128
K = 64
B = 4
NPER = N // B
NC, NS = 2, 16
NW = NC * NS
CH = 128
EPAD = 327680
EB_TILE = EPAD // NW
EB_CHUNKS = EB_TILE // CH
ED_TILE = EPAD // NS
ED_CHUNKS = ED_TILE // CH
NTR = 10112
TRASH = 10000


def _project_body(h_ref, w_ref, o_ref):
    mm = jnp.dot(h_ref[...], w_ref[...], preferred_element_type=jnp.float32)
    blk = mm.shape[0]
    o_ref[...] = jnp.concatenate(
        [mm, jnp.ones((blk, 1), jnp.float32),
         jnp.zeros((blk, 127 - K), jnp.float32)], axis=1)


def _project(h, w):
    blk = 1000
    return pl.pallas_call(
        _project_body,
        grid=(N // blk,),
        in_specs=[
            pl.BlockSpec((blk, D), lambda i: (i, 0)),
            pl.BlockSpec((D, K), lambda i: (0, 0)),
        ],
        out_specs=pl.BlockSpec((blk, 128), lambda i: (i, 0)),
        out_shape=jax.ShapeDtypeStruct((N, 128), jnp.float32),
    )(h, w)


def _segsum_body(src_hbm, dst_hbm, p_hbm, z_hbm, agg_out,
                 sidx, didx, rows, acc_sh, sem):
    c = lax.axis_index("c")
    s = lax.axis_index("s")

    zrows = NTR // NS
    pltpu.sync_copy(z_hbm.at[pl.ds(s * zrows, zrows)],
                    acc_sh.at[pl.ds(s * zrows, zrows)])
    plsc.subcore_barrier()

    base = (c * NS + s) * EB_TILE

    def chunk(i, carry):
        e0 = base + i * CH
        pltpu.sync_copy(src_hbm.at[pl.ds(e0, CH)], sidx)
        pltpu.sync_copy(dst_hbm.at[pl.ds(e0, CH)], didx)
        pltpu.async_copy(p_hbm.at[sidx], rows, sem).wait()
        pltpu.sync_copy(rows, acc_sh.at[didx], add=True)
        return carry

    lax.fori_loop(0, EB_CHUNKS, chunk, 0)
    plsc.subcore_barrier()

    pltpu.sync_copy(acc_sh.at[pl.ds(s * zrows, zrows)],
                    agg_out.at[c, pl.ds(s * zrows, zrows)])


def _segsum(src, dst, p, z):
    mesh = plsc.VectorSubcoreMesh(core_axis_name="c", subcore_axis_name="s",
                                  num_cores=NC, num_subcores=NS)
    return pl.kernel(
        _segsum_body,
        out_type=jax.ShapeDtypeStruct((NC, NTR, 128), jnp.float32),
        mesh=mesh,
        scratch_types=[
            pltpu.VMEM((CH,), jnp.int32),
            pltpu.VMEM((CH,), jnp.int32),
            pltpu.VMEM((CH, 128), jnp.float32),
            pltpu.VMEM_SHARED((NTR, 128), jnp.float32),
            pltpu.SemaphoreType.DMA,
        ],
        compiler_params=pltpu.CompilerParams(use_tc_tiling_on_sc=True),
    )(src, dst, p, z)


def _softmax_body(h_ref, w_ref, agg_ref, s_ref, t_ref, reg_ref):
    i = pl.program_id(0)
    n = pl.num_programs(0)
    blk = h_ref.shape[0]
    q = jnp.dot(h_ref[...], w_ref[...], preferred_element_type=jnp.float32)
    aggs = agg_ref[0, :, 0:K] + agg_ref[1, :, 0:K]
    deg = agg_ref[0, :, K:K + 1] + agg_ref[1, :, K:K + 1]
    logits = q + aggs / jnp.maximum(deg, 1.0)
    m = jnp.max(logits, axis=1, keepdims=True)
    ex = jnp.exp(logits - m)
    sm = ex / jnp.sum(ex, axis=1, keepdims=True)
    s_ref[...] = sm
    t_ref[...] = jnp.concatenate(
        [sm, jnp.zeros((blk, 128), jnp.float32), sm], axis=1)
    part = jnp.sum(sm * sm - sm * jnp.log(sm + 1e-12)).reshape(1, 1)
    prev = jnp.where(i == 0, jnp.zeros((1, 1), jnp.float32), reg_ref[...])
    tot = prev + part
    reg_ref[...] = jnp.where(i == n - 1, tot / N, tot)


def _softmax(h, w_self, agg):
    blk = 1000
    return pl.pallas_call(
        _softmax_body,
        grid=(N // blk,),
        in_specs=[
            pl.BlockSpec((blk, D), lambda i: (i, 0)),
            pl.BlockSpec((D, K), lambda i: (0, 0)),
            pl.BlockSpec((NC, blk, 128), lambda i: (0, i, 0)),
        ],
        out_specs=[
            pl.BlockSpec((blk, K), lambda i: (i, 0)),
            pl.BlockSpec((blk, 4 * K), lambda i: (i, 0)),
            pl.BlockSpec((1, 1), lambda i: (0, 0)),
        ],
        out_shape=[
            jax.ShapeDtypeStruct((N, K), jnp.float32),
            jax.ShapeDtypeStruct((N, 4 * K), jnp.float32),
            jax.ShapeDtypeStruct((1, 1), jnp.float32),
        ],
    )(h, w_self, agg)


def _as_body(src_hbm, dst_hbm, t_hbm, z_hbm, as_out,
             sidx, didx, gidx, lidx, rows, acc_sh, sem):
    c = lax.axis_index("c")
    s = lax.axis_index("s")

    zrows = NTR // NS
    pltpu.sync_copy(z_hbm.at[pl.ds(s * zrows, zrows)],
                    acc_sh.at[pl.ds(s * zrows, zrows)])
    plsc.subcore_barrier()

    base = s * ED_TILE
    lo = 2 * c * NPER
    mid = lo + NPER
    hi = lo + 2 * NPER

    def chunk(i, carry):
        e0 = base + i * CH
        pltpu.sync_copy(src_hbm.at[pl.ds(e0, CH)], sidx)
        pltpu.sync_copy(dst_hbm.at[pl.ds(e0, CH)], didx)
        for j in range(CH // 16):
            sl = pl.ds(j * 16, 16)
            sv = sidx[sl]
            dv = didx[sl]
            gidx[sl] = 2 * sv + (dv & 1)
            valid = (sv >= lo) & (sv < hi) & (dv < N)
            half_off = jnp.where(sv >= mid, N // 2, 0)
            lidx[sl] = jnp.where(valid, half_off + (dv >> 1), TRASH)
        pltpu.async_copy(t_hbm.at[gidx], rows, sem).wait()
        pltpu.sync_copy(rows, acc_sh.at[lidx], add=True)
        return carry

    lax.fori_loop(0, ED_CHUNKS, chunk, 0)
    plsc.subcore_barrier()

    pltpu.sync_copy(acc_sh.at[pl.ds(s * zrows, zrows)],
                    as_out.at[c, pl.ds(s * zrows, zrows)])


def _as_scatter(src, dst, t2, z):
    mesh = plsc.VectorSubcoreMesh(core_axis_name="c", subcore_axis_name="s",
                                  num_cores=NC, num_subcores=NS)
    return pl.kernel(
        _as_body,
        out_type=jax.ShapeDtypeStruct((NC, NTR, 128), jnp.float32),
        mesh=mesh,
        scratch_types=[
            pltpu.VMEM((CH,), jnp.int32),
            pltpu.VMEM((CH,), jnp.int32),
            pltpu.VMEM((CH,), jnp.int32),
            pltpu.VMEM((CH,), jnp.int32),
            pltpu.VMEM((CH, 128), jnp.float32),
            pltpu.VMEM_SHARED((NTR, 128), jnp.float32),
            pltpu.SemaphoreType.DMA,
        ],
        compiler_params=pltpu.CompilerParams(use_tc_tiling_on_sc=True),
    )(src, dst, t2, z)


def _adj_body(s_ref, as_ref, o_ref):
    sg = s_ref[0]
    blocks = []
    for gs in range(B):
        a = as_ref[gs, 0]
        blocks.append(lax.dot_general(
            sg, a, (((0,), (0,)), ((), ())),
            preferred_element_type=jnp.float32))
    o_ref[...] = jnp.concatenate(blocks, axis=1)


def _adj(s_r, as_r):
    return pl.pallas_call(
        _adj_body,
        grid=(B,),
        in_specs=[
            pl.BlockSpec((1, NPER, K), lambda gd: (gd, 0, 0)),
            pl.BlockSpec((B, 1, NPER, K), lambda gd: (0, gd, 0, 0)),
        ],
        out_specs=pl.BlockSpec((K, B * K), lambda gd: (gd, 0)),
        out_shape=jax.ShapeDtypeStruct((B * K, B * K), jnp.float32),
    )(s_r, as_r)


@jax.jit
def kernel(h, edge_index, W_self, W_neigh):
    src = edge_index[0]
    dst = edge_index[1]
    npad = EPAD - E
    src_pad = jnp.concatenate([src, jnp.zeros((npad,), jnp.int32)])
    dst_pad = jnp.concatenate([dst, jnp.full((npad,), N, jnp.int32)])
    z = jnp.zeros((NTR, 128), jnp.float32)

    p128 = _project(h, W_neigh)
    agg = _segsum(src_pad, dst_pad, p128, z)
    s_mat, t_mat, reg = _softmax(h, W_self, agg)
    t2 = t_mat.reshape(2 * N, 128)
    as4 = _as_scatter(src_pad, dst_pad, t2, z)
    flat = as4.reshape(NC, 2 * NTR, K)
    s_r = s_mat.reshape(B, NPER, K)
    as_r = flat[:, :2 * N].reshape(NC, 2, B, NPER, K).reshape(B, B, NPER, K)
    adj_new = _adj(s_r, as_r)
    return adj_new, reg[0, 0]


# speedup vs baseline: 2.7053x; 1.1258x over previous
import jax
import jax.numpy as jnp
from jax import lax
from jax.experimental import pallas as pl
from jax.experimental.pallas import tpu as pltpu
from jax.experimental.pallas import tpu_sc as plsc

N = 10000
E = 320000
D = 128
K = 64
B = 4
NPER = N // B
NC, NS = 2, 16
NW = NC * NS
CH = 128
EPAD = 327680
EB_TILE = EPAD // NW
EB_CHUNKS = EB_TILE // CH
ED_TILE = EPAD // NS
ED_CHUNKS = ED_TILE // CH
NTR = 10112
TRASH = 10000


def _project_body(h_ref, w_ref, o_ref):
    mm = jnp.dot(h_ref[...], w_ref[...], preferred_element_type=jnp.float32)
    blk = mm.shape[0]
    o_ref[...] = jnp.concatenate(
        [mm, jnp.ones((blk, 1), jnp.float32),
         jnp.zeros((blk, 127 - K), jnp.float32)], axis=1)


def _project(h, w):
    blk = 1000
    return pl.pallas_call(
        _project_body,
        grid=(N // blk,),
        in_specs=[
            pl.BlockSpec((blk, D), lambda i: (i, 0)),
            pl.BlockSpec((D, K), lambda i: (0, 0)),
        ],
        out_specs=pl.BlockSpec((blk, 128), lambda i: (i, 0)),
        out_shape=jax.ShapeDtypeStruct((N, 128), jnp.float32),
    )(h, w)


def _edge_sweep(n_chunks, base, edges_hbm, eidx, isem,
                issue_gather, wait_gather, issue_scatter, wait_scatter,
                compute=None):
    T = n_chunks // 4

    def idx_load(chunk, u):
        pltpu.async_copy(
            edges_hbm.at[:, pl.ds(base + chunk * CH, CH)], eidx[u], isem[u])

    def wait_idx(u):
        pltpu.make_async_copy(
            edges_hbm.at[:, pl.ds(0, CH)], eidx[u], isem[u]).wait()

    idx_load(0, 0)
    idx_load(1, 1)
    wait_idx(0)
    if compute is not None:
        compute(0, 0)
    issue_gather(0, 0)

    def step(t, carry):
        for u in range(4):
            r, r1, u1, u2 = u % 2, (u + 1) % 2, (u + 1) % 4, (u + 2) % 4
            wait_gather(r)
            issue_scatter(u, r)
            if u < 3:
                wait_idx(u1)
                if u == 0:
                    @pl.when(t >= 1)
                    def _():
                        wait_scatter(r1)
                else:
                    wait_scatter(r1)
                if compute is not None:
                    compute(u1, r1)
                issue_gather(u1, r1)
            else:
                @pl.when(t < T - 1)
                def _():
                    wait_idx(u1)
                    wait_scatter(r1)
                    if compute is not None:
                        compute(u1, r1)
                    issue_gather(u1, r1)
            if u < 2:
                idx_load(4 * t + u + 2, u2)
            else:
                @pl.when(t < T - 1)
                def _():
                    idx_load(4 * t + u + 2, u2)
        return carry

    lax.fori_loop(0, T, step, 0)
    wait_scatter(0)
    wait_scatter(1)


def _segsum_body(edges_hbm, p_hbm, z_hbm, agg_out,
                 e0v, e1v, e2v, e3v, rows0, rows1, acc_sh,
                 is0, is1, is2, is3, gs0, gs1, ss0, ss1):
    c = lax.axis_index("c")
    s = lax.axis_index("s")

    zrows = NTR // NS
    pltpu.sync_copy(z_hbm.at[pl.ds(s * zrows, zrows)],
                    acc_sh.at[pl.ds(s * zrows, zrows)])
    plsc.subcore_barrier()

    base = (c * NS + s) * EB_TILE
    eidx = [e0v, e1v, e2v, e3v]
    rows = [rows0, rows1]
    isem = [is0, is1, is2, is3]
    gsem = [gs0, gs1]
    ssem = [ss0, ss1]

    def issue_gather(u, r):
        pltpu.async_copy(p_hbm.at[eidx[u].at[0]], rows[r], gsem[r])

    def wait_gather(r):
        pltpu.make_async_copy(p_hbm.at[eidx[0].at[0]], rows[r],
                              gsem[r]).wait()

    def issue_scatter(u, r):
        pltpu.async_copy(rows[r], acc_sh.at[eidx[u].at[1]], ssem[r],
                         add=True)

    def wait_scatter(r):
        pltpu.make_async_copy(rows[r], acc_sh.at[eidx[0].at[1]],
                              ssem[r]).wait()

    _edge_sweep(EB_CHUNKS, base, edges_hbm, eidx, isem,
                issue_gather, wait_gather, issue_scatter, wait_scatter)

    plsc.subcore_barrier()
    pltpu.sync_copy(acc_sh.at[pl.ds(s * zrows, zrows)],
                    agg_out.at[c, pl.ds(s * zrows, zrows)])


def _segsum(edges, p, z):
    mesh = plsc.VectorSubcoreMesh(core_axis_name="c", subcore_axis_name="s",
                                  num_cores=NC, num_subcores=NS)
    return pl.kernel(
        _segsum_body,
        out_type=jax.ShapeDtypeStruct((NC, NTR, 128), jnp.float32),
        mesh=mesh,
        scratch_types=[
            pltpu.VMEM((2, CH), jnp.int32),
            pltpu.VMEM((2, CH), jnp.int32),
            pltpu.VMEM((2, CH), jnp.int32),
            pltpu.VMEM((2, CH), jnp.int32),
            pltpu.VMEM((CH, 128), jnp.float32),
            pltpu.VMEM((CH, 128), jnp.float32),
            pltpu.VMEM_SHARED((NTR, 128), jnp.float32),
            pltpu.SemaphoreType.DMA,
            pltpu.SemaphoreType.DMA,
            pltpu.SemaphoreType.DMA,
            pltpu.SemaphoreType.DMA,
            pltpu.SemaphoreType.DMA,
            pltpu.SemaphoreType.DMA,
            pltpu.SemaphoreType.DMA,
            pltpu.SemaphoreType.DMA,
        ],
        compiler_params=pltpu.CompilerParams(use_tc_tiling_on_sc=True),
    )(edges, p, z)


def _softmax_body(h_ref, w_ref, agg_ref, s_ref, t_ref, reg_ref):
    i = pl.program_id(0)
    n = pl.num_programs(0)
    blk = h_ref.shape[0]
    q = jnp.dot(h_ref[...], w_ref[...], preferred_element_type=jnp.float32)
    aggs = agg_ref[0, :, 0:K] + agg_ref[1, :, 0:K]
    deg = agg_ref[0, :, K:K + 1] + agg_ref[1, :, K:K + 1]
    logits = q + aggs / jnp.maximum(deg, 1.0)
    m = jnp.max(logits, axis=1, keepdims=True)
    ex = jnp.exp(logits - m)
    sm = ex / jnp.sum(ex, axis=1, keepdims=True)
    s_ref[...] = sm
    t_ref[...] = jnp.concatenate(
        [sm, jnp.zeros((blk, 128), jnp.float32), sm], axis=1)
    part = jnp.sum(sm * sm - sm * jnp.log(sm + 1e-12)).reshape(1, 1)
    prev = jnp.where(i == 0, jnp.zeros((1, 1), jnp.float32), reg_ref[...])
    tot = prev + part
    reg_ref[...] = jnp.where(i == n - 1, tot / N, tot)


def _softmax(h, w_self, agg):
    blk = 1000
    return pl.pallas_call(
        _softmax_body,
        grid=(N // blk,),
        in_specs=[
            pl.BlockSpec((blk, D), lambda i: (i, 0)),
            pl.BlockSpec((D, K), lambda i: (0, 0)),
            pl.BlockSpec((NC, blk, 128), lambda i: (0, i, 0)),
        ],
        out_specs=[
            pl.BlockSpec((blk, K), lambda i: (i, 0)),
            pl.BlockSpec((blk, 4 * K), lambda i: (i, 0)),
            pl.BlockSpec((1, 1), lambda i: (0, 0)),
        ],
        out_shape=[
            jax.ShapeDtypeStruct((N, K), jnp.float32),
            jax.ShapeDtypeStruct((N, 4 * K), jnp.float32),
            jax.ShapeDtypeStruct((1, 1), jnp.float32),
        ],
    )(h, w_self, agg)


def _as_body(edges_hbm, t2_hbm, z_hbm, as_out,
             e0v, e1v, e2v, e3v, g0v, g1v, l0v, l1v, rows0, rows1, acc_sh,
             is0, is1, is2, is3, gs0, gs1, ss0, ss1):
    c = lax.axis_index("c")
    s = lax.axis_index("s")

    zrows = NTR // NS
    pltpu.sync_copy(z_hbm.at[pl.ds(s * zrows, zrows)],
                    acc_sh.at[pl.ds(s * zrows, zrows)])
    plsc.subcore_barrier()

    base = s * ED_TILE
    lo = 2 * c * NPER
    mid = lo + NPER
    hi = lo + 2 * NPER

    eidx = [e0v, e1v, e2v, e3v]
    gidx = [g0v, g1v]
    lidx = [l0v, l1v]
    rows = [rows0, rows1]
    isem = [is0, is1, is2, is3]
    gsem = [gs0, gs1]
    ssem = [ss0, ss1]

    def compute(u, r):
        for j in range(CH // 16):
            sl = pl.ds(j * 16, 16)
            sv = eidx[u][0, sl]
            dv = eidx[u][1, sl]
            gidx[r][sl] = 2 * sv + (dv & 1)
            valid = (sv >= lo) & (sv < hi) & (dv < N)
            half_off = jnp.where(sv >= mid, N // 2, 0)
            lidx[r][sl] = jnp.where(valid, half_off + (dv >> 1), TRASH)

    def issue_gather(u, r):
        pltpu.async_copy(t2_hbm.at[gidx[r]], rows[r], gsem[r])

    def wait_gather(r):
        pltpu.make_async_copy(t2_hbm.at[gidx[r]], rows[r], gsem[r]).wait()

    def issue_scatter(u, r):
        pltpu.async_copy(rows[r], acc_sh.at[lidx[r]], ssem[r], add=True)

    def wait_scatter(r):
        pltpu.make_async_copy(rows[r], acc_sh.at[lidx[r]], ssem[r]).wait()

    _edge_sweep(ED_CHUNKS, base, edges_hbm, eidx, isem,
                issue_gather, wait_gather, issue_scatter, wait_scatter,
                compute=compute)

    plsc.subcore_barrier()
    pltpu.sync_copy(acc_sh.at[pl.ds(s * zrows, zrows)],
                    as_out.at[c, pl.ds(s * zrows, zrows)])


def _as_scatter(edges, t2, z):
    mesh = plsc.VectorSubcoreMesh(core_axis_name="c", subcore_axis_name="s",
                                  num_cores=NC, num_subcores=NS)
    return pl.kernel(
        _as_body,
        out_type=jax.ShapeDtypeStruct((NC, NTR, 128), jnp.float32),
        mesh=mesh,
        scratch_types=[
            pltpu.VMEM((2, CH), jnp.int32),
            pltpu.VMEM((2, CH), jnp.int32),
            pltpu.VMEM((2, CH), jnp.int32),
            pltpu.VMEM((2, CH), jnp.int32),
            pltpu.VMEM((CH,), jnp.int32),
            pltpu.VMEM((CH,), jnp.int32),
            pltpu.VMEM((CH,), jnp.int32),
            pltpu.VMEM((CH,), jnp.int32),
            pltpu.VMEM((CH, 128), jnp.float32),
            pltpu.VMEM((CH, 128), jnp.float32),
            pltpu.VMEM_SHARED((NTR, 128), jnp.float32),
            pltpu.SemaphoreType.DMA,
            pltpu.SemaphoreType.DMA,
            pltpu.SemaphoreType.DMA,
            pltpu.SemaphoreType.DMA,
            pltpu.SemaphoreType.DMA,
            pltpu.SemaphoreType.DMA,
            pltpu.SemaphoreType.DMA,
            pltpu.SemaphoreType.DMA,
        ],
        compiler_params=pltpu.CompilerParams(use_tc_tiling_on_sc=True),
    )(edges, t2, z)


def _adj_body(s_ref, as_ref, o_ref):
    sg = s_ref[0]
    blocks = []
    for gs in range(B):
        a = as_ref[gs, 0]
        blocks.append(lax.dot_general(
            sg, a, (((0,), (0,)), ((), ())),
            preferred_element_type=jnp.float32))
    o_ref[...] = jnp.concatenate(blocks, axis=1)


def _adj(s_r, as_r):
    return pl.pallas_call(
        _adj_body,
        grid=(B,),
        in_specs=[
            pl.BlockSpec((1, NPER, K), lambda gd: (gd, 0, 0)),
            pl.BlockSpec((B, 1, NPER, K), lambda gd: (0, gd, 0, 0)),
        ],
        out_specs=pl.BlockSpec((K, B * K), lambda gd: (gd, 0)),
        out_shape=jax.ShapeDtypeStruct((B * K, B * K), jnp.float32),
    )(s_r, as_r)


@jax.jit
def kernel(h, edge_index, W_self, W_neigh):
    npad = EPAD - E
    pad = jnp.concatenate(
        [jnp.zeros((1, npad), jnp.int32),
         jnp.full((1, npad), N, jnp.int32)], axis=0)
    edges = jnp.concatenate([edge_index, pad], axis=1)
    z = jnp.zeros((NTR, 128), jnp.float32)

    p128 = _project(h, W_neigh)
    agg = _segsum(edges, p128, z)
    s_mat, t_mat, reg = _softmax(h, W_self, agg)
    t2 = t_mat.reshape(2 * N, 128)
    as4 = _as_scatter(edges, t2, z)
    flat = as4.reshape(NC, 2 * NTR, K)
    s_r = s_mat.reshape(B, NPER, K)
    as_r = flat[:, :2 * N].reshape(NC, 2, B, NPER, K).reshape(B, B, NPER, K)
    adj_new = _adj(s_r, as_r)
    return adj_new, reg[0, 0]


# speedup vs baseline: 2.7212x; 1.0059x over previous
import jax
import jax.numpy as jnp
from jax import lax
from jax.experimental import pallas as pl
from jax.experimental.pallas import tpu as pltpu
from jax.experimental.pallas import tpu_sc as plsc

N = 10000
E = 320000
D = 128
K = 64
B = 4
NPER = N // B
NC, NS = 2, 16
NW = NC * NS
CH = 128
EPAD = 327680
EB_TILE = EPAD // NW
EB_CHUNKS = EB_TILE // CH
ED_TILE = EPAD // NS
ED_CHUNKS = ED_TILE // CH
NTR = 10112
TRASH = 10000


def _project_body(h_ref, w_ref, o_ref):
    mm = jnp.dot(h_ref[...], w_ref[...], preferred_element_type=jnp.float32)
    blk = mm.shape[0]
    o_ref[...] = jnp.concatenate(
        [mm, jnp.ones((blk, 1), jnp.float32),
         jnp.zeros((blk, 127 - K), jnp.float32)], axis=1)


def _project(h, w):
    blk = 1000
    return pl.pallas_call(
        _project_body,
        grid=(N // blk,),
        in_specs=[
            pl.BlockSpec((blk, D), lambda i: (i, 0)),
            pl.BlockSpec((D, K), lambda i: (0, 0)),
        ],
        out_specs=pl.BlockSpec((blk, 128), lambda i: (i, 0)),
        out_shape=jax.ShapeDtypeStruct((N, 128), jnp.float32),
    )(h, w)


def _edge_sweep(n_chunks, base, edges_hbm, eidx, isem,
                issue_gather, wait_gather, issue_scatter, wait_scatter,
                compute=None):
    T = n_chunks // 4

    def idx_load(chunk, u):
        pltpu.async_copy(
            edges_hbm.at[:, pl.ds(base + chunk * CH, CH)], eidx[u], isem[u])

    def wait_idx(u):
        pltpu.make_async_copy(
            edges_hbm.at[:, pl.ds(0, CH)], eidx[u], isem[u]).wait()

    idx_load(0, 0)
    idx_load(1, 1)
    wait_idx(0)
    if compute is not None:
        compute(0, 0)
    issue_gather(0, 0)

    def step(t, carry):
        for u in range(4):
            r, r1, u1, u2 = u % 2, (u + 1) % 2, (u + 1) % 4, (u + 2) % 4
            wait_gather(r)
            issue_scatter(u, r)
            if u < 3:
                wait_idx(u1)
                if u == 0:
                    @pl.when(t >= 1)
                    def _():
                        wait_scatter(r1)
                else:
                    wait_scatter(r1)
                if compute is not None:
                    compute(u1, r1)
                issue_gather(u1, r1)
            else:
                @pl.when(t < T - 1)
                def _():
                    wait_idx(u1)
                    wait_scatter(r1)
                    if compute is not None:
                        compute(u1, r1)
                    issue_gather(u1, r1)
            if u < 2:
                idx_load(4 * t + u + 2, u2)
            else:
                @pl.when(t < T - 1)
                def _():
                    idx_load(4 * t + u + 2, u2)
        return carry

    lax.fori_loop(0, T, step, 0)
    wait_scatter(0)
    wait_scatter(1)


def _segsum_body(edges_hbm, p_hbm, z_hbm, agg_out,
                 e0v, e1v, e2v, e3v, rows0, rows1, acc_sh,
                 is0, is1, is2, is3, gs0, gs1, ss0, ss1):
    c = lax.axis_index("c")
    s = lax.axis_index("s")

    zrows = NTR // NS
    pltpu.sync_copy(z_hbm.at[pl.ds(s * zrows, zrows)],
                    acc_sh.at[pl.ds(s * zrows, zrows)])
    plsc.subcore_barrier()

    base = (c * NS + s) * EB_TILE
    eidx = [e0v, e1v, e2v, e3v]
    rows = [rows0, rows1]
    isem = [is0, is1, is2, is3]
    gsem = [gs0, gs1]
    ssem = [ss0, ss1]

    def issue_gather(u, r):
        pltpu.async_copy(p_hbm.at[eidx[u].at[0]], rows[r], gsem[r])

    def wait_gather(r):
        pltpu.make_async_copy(p_hbm.at[eidx[0].at[0]], rows[r],
                              gsem[r]).wait()

    def issue_scatter(u, r):
        pltpu.async_copy(rows[r], acc_sh.at[eidx[u].at[1]], ssem[r],
                         add=True)

    def wait_scatter(r):
        pltpu.make_async_copy(rows[r], acc_sh.at[eidx[0].at[1]],
                              ssem[r]).wait()

    _edge_sweep(EB_CHUNKS, base, edges_hbm, eidx, isem,
                issue_gather, wait_gather, issue_scatter, wait_scatter)

    plsc.subcore_barrier()
    pltpu.sync_copy(acc_sh.at[pl.ds(s * zrows, zrows)],
                    agg_out.at[c, pl.ds(s * zrows, zrows)])


def _segsum(edges, p, z):
    mesh = plsc.VectorSubcoreMesh(core_axis_name="c", subcore_axis_name="s",
                                  num_cores=NC, num_subcores=NS)
    return pl.kernel(
        _segsum_body,
        out_type=jax.ShapeDtypeStruct((NC, NTR, 128), jnp.float32),
        mesh=mesh,
        scratch_types=[
            pltpu.VMEM((2, CH), jnp.int32),
            pltpu.VMEM((2, CH), jnp.int32),
            pltpu.VMEM((2, CH), jnp.int32),
            pltpu.VMEM((2, CH), jnp.int32),
            pltpu.VMEM((CH, 128), jnp.float32),
            pltpu.VMEM((CH, 128), jnp.float32),
            pltpu.VMEM_SHARED((NTR, 128), jnp.float32),
            pltpu.SemaphoreType.DMA,
            pltpu.SemaphoreType.DMA,
            pltpu.SemaphoreType.DMA,
            pltpu.SemaphoreType.DMA,
            pltpu.SemaphoreType.DMA,
            pltpu.SemaphoreType.DMA,
            pltpu.SemaphoreType.DMA,
            pltpu.SemaphoreType.DMA,
        ],
        compiler_params=pltpu.CompilerParams(use_tc_tiling_on_sc=True),
    )(edges, p, z)


def _softmax_body(h_ref, w_ref, agg_ref, s_ref, t_ref, reg_ref):
    i = pl.program_id(0)
    n = pl.num_programs(0)
    blk = h_ref.shape[0]
    q = jnp.dot(h_ref[...], w_ref[...], preferred_element_type=jnp.float32)
    aggs = agg_ref[0, :, 0:K] + agg_ref[1, :, 0:K]
    deg = agg_ref[0, :, K:K + 1] + agg_ref[1, :, K:K + 1]
    logits = q + aggs / jnp.maximum(deg, 1.0)
    m = jnp.max(logits, axis=1, keepdims=True)
    ex = jnp.exp(logits - m)
    sm = ex / jnp.sum(ex, axis=1, keepdims=True)
    s_ref[...] = sm
    t_ref[...] = jnp.concatenate(
        [sm, jnp.zeros((blk, 128), jnp.float32), sm], axis=1)
    part = jnp.sum(sm * sm - sm * jnp.log(sm + 1e-12)).reshape(1, 1)
    prev = jnp.where(i == 0, jnp.zeros((1, 1), jnp.float32), reg_ref[...])
    tot = prev + part
    reg_ref[...] = jnp.where(i == n - 1, tot / N, tot)


def _softmax(h, w_self, agg):
    blk = 1000
    return pl.pallas_call(
        _softmax_body,
        grid=(N // blk,),
        in_specs=[
            pl.BlockSpec((blk, D), lambda i: (i, 0)),
            pl.BlockSpec((D, K), lambda i: (0, 0)),
            pl.BlockSpec((NC, blk, 128), lambda i: (0, i, 0)),
        ],
        out_specs=[
            pl.BlockSpec((blk, K), lambda i: (i, 0)),
            pl.BlockSpec((blk, 4 * K), lambda i: (i, 0)),
            pl.BlockSpec((1, 1), lambda i: (0, 0)),
        ],
        out_shape=[
            jax.ShapeDtypeStruct((N, K), jnp.float32),
            jax.ShapeDtypeStruct((N, 4 * K), jnp.float32),
            jax.ShapeDtypeStruct((1, 1), jnp.float32),
        ],
    )(h, w_self, agg)


def _as_body(edges_hbm, t2_hbm, z_hbm, as_out,
             e0v, e1v, e2v, e3v, g0v, g1v, l0v, l1v, rows0, rows1, acc_sh,
             is0, is1, is2, is3, gs0, gs1, ss0, ss1):
    c = lax.axis_index("c")
    s = lax.axis_index("s")

    zrows = NTR // NS
    pltpu.sync_copy(z_hbm.at[pl.ds(s * zrows, zrows)],
                    acc_sh.at[pl.ds(s * zrows, zrows)])
    plsc.subcore_barrier()

    base = s * ED_TILE
    lo = 2 * c * NPER
    mid = lo + NPER
    hi = lo + 2 * NPER

    eidx = [e0v, e1v, e2v, e3v]
    gidx = [g0v, g1v]
    lidx = [l0v, l1v]
    rows = [rows0, rows1]
    isem = [is0, is1, is2, is3]
    gsem = [gs0, gs1]
    ssem = [ss0, ss1]

    def compute(u, r):
        for j in range(CH // 16):
            sl = pl.ds(j * 16, 16)
            sv = eidx[u][0, sl]
            dv = eidx[u][1, sl]
            gidx[r][sl] = 2 * sv + (dv & 1)
            valid = (sv >= lo) & (sv < hi) & (dv < N)
            half_off = jnp.where(sv >= mid, N // 2, 0)
            lidx[r][sl] = jnp.where(valid, half_off + (dv >> 1),
                                    TRASH + (dv & 63))

    def issue_gather(u, r):
        pltpu.async_copy(t2_hbm.at[gidx[r]], rows[r], gsem[r])

    def wait_gather(r):
        pltpu.make_async_copy(t2_hbm.at[gidx[r]], rows[r], gsem[r]).wait()

    def issue_scatter(u, r):
        pltpu.async_copy(rows[r], acc_sh.at[lidx[r]], ssem[r], add=True)

    def wait_scatter(r):
        pltpu.make_async_copy(rows[r], acc_sh.at[lidx[r]], ssem[r]).wait()

    _edge_sweep(ED_CHUNKS, base, edges_hbm, eidx, isem,
                issue_gather, wait_gather, issue_scatter, wait_scatter,
                compute=compute)

    plsc.subcore_barrier()
    pltpu.sync_copy(acc_sh.at[pl.ds(s * zrows, zrows)],
                    as_out.at[c, pl.ds(s * zrows, zrows)])


def _as_scatter(edges, t2, z):
    mesh = plsc.VectorSubcoreMesh(core_axis_name="c", subcore_axis_name="s",
                                  num_cores=NC, num_subcores=NS)
    return pl.kernel(
        _as_body,
        out_type=jax.ShapeDtypeStruct((NC, NTR, 128), jnp.float32),
        mesh=mesh,
        scratch_types=[
            pltpu.VMEM((2, CH), jnp.int32),
            pltpu.VMEM((2, CH), jnp.int32),
            pltpu.VMEM((2, CH), jnp.int32),
            pltpu.VMEM((2, CH), jnp.int32),
            pltpu.VMEM((CH,), jnp.int32),
            pltpu.VMEM((CH,), jnp.int32),
            pltpu.VMEM((CH,), jnp.int32),
            pltpu.VMEM((CH,), jnp.int32),
            pltpu.VMEM((CH, 128), jnp.float32),
            pltpu.VMEM((CH, 128), jnp.float32),
            pltpu.VMEM_SHARED((NTR, 128), jnp.float32),
            pltpu.SemaphoreType.DMA,
            pltpu.SemaphoreType.DMA,
            pltpu.SemaphoreType.DMA,
            pltpu.SemaphoreType.DMA,
            pltpu.SemaphoreType.DMA,
            pltpu.SemaphoreType.DMA,
            pltpu.SemaphoreType.DMA,
            pltpu.SemaphoreType.DMA,
        ],
        compiler_params=pltpu.CompilerParams(use_tc_tiling_on_sc=True),
    )(edges, t2, z)


def _adj_body(s_ref, as_ref, o_ref):
    sg = s_ref[0]
    blocks = []
    for gs in range(B):
        a = as_ref[gs, 0]
        blocks.append(lax.dot_general(
            sg, a, (((0,), (0,)), ((), ())),
            preferred_element_type=jnp.float32))
    o_ref[...] = jnp.concatenate(blocks, axis=1)


def _adj(s_r, as_r):
    return pl.pallas_call(
        _adj_body,
        grid=(B,),
        in_specs=[
            pl.BlockSpec((1, NPER, K), lambda gd: (gd, 0, 0)),
            pl.BlockSpec((B, 1, NPER, K), lambda gd: (0, gd, 0, 0)),
        ],
        out_specs=pl.BlockSpec((K, B * K), lambda gd: (gd, 0)),
        out_shape=jax.ShapeDtypeStruct((B * K, B * K), jnp.float32),
    )(s_r, as_r)


@jax.jit
def kernel(h, edge_index, W_self, W_neigh):
    npad = EPAD - E
    pad = jnp.concatenate(
        [jnp.zeros((1, npad), jnp.int32),
         jnp.full((1, npad), N, jnp.int32)], axis=0)
    edges = jnp.concatenate([edge_index, pad], axis=1)
    z = jnp.zeros((NTR, 128), jnp.float32)

    p128 = _project(h, W_neigh)
    agg = _segsum(edges, p128, z)
    s_mat, t_mat, reg = _softmax(h, W_self, agg)
    t2 = t_mat.reshape(2 * N, 128)
    as4 = _as_scatter(edges, t2, z)
    flat = as4.reshape(NC, 2 * NTR, K)
    s_r = s_mat.reshape(B, NPER, K)
    as_r = flat[:, :2 * N].reshape(NC, 2, B, NPER, K).reshape(B, B, NPER, K)
    adj_new = _adj(s_r, as_r)
    return adj_new, reg[0, 0]


# speedup vs baseline: 3.0163x; 1.1084x over previous
import jax
import jax.numpy as jnp
from jax import lax
from jax.experimental import pallas as pl
from jax.experimental.pallas import tpu as pltpu
from jax.experimental.pallas import tpu_sc as plsc

N = 10000
E = 320000
D = 128
K = 64
B = 4
NPER = N // B
NC, NS = 2, 16
NW = NC * NS
CH = 128
EPAD = 327680
EB_TILE = EPAD // NW
EB_CHUNKS = EB_TILE // CH
ED_TILE = EPAD // NS
ED_CHUNKS = ED_TILE // CH
NTR = 10112
TRASH = 10000


def _project_body(h_ref, w_ref, o_ref):
    mm = jnp.dot(h_ref[...], w_ref[...], preferred_element_type=jnp.float32)
    blk = mm.shape[0]
    o_ref[...] = jnp.concatenate(
        [mm, jnp.ones((blk, 1), jnp.float32),
         jnp.zeros((blk, 127 - K), jnp.float32)], axis=1)


def _project(h, w):
    blk = 1000
    return pl.pallas_call(
        _project_body,
        grid=(N // blk,),
        in_specs=[
            pl.BlockSpec((blk, D), lambda i: (i, 0)),
            pl.BlockSpec((D, K), lambda i: (0, 0)),
        ],
        out_specs=pl.BlockSpec((blk, 128), lambda i: (i, 0)),
        out_shape=jax.ShapeDtypeStruct((N, 128), jnp.float32),
    )(h, w)


def _edge_sweep(n_chunks, base, edges_hbm, eidx, isem,
                issue_gather, wait_gather, issue_scatter, wait_scatter,
                compute=None):
    T = n_chunks // 4

    def idx_load(chunk, u):
        pltpu.async_copy(
            edges_hbm.at[:, pl.ds(base + chunk * CH, CH)], eidx[u], isem[u])

    def wait_idx(u):
        pltpu.make_async_copy(
            edges_hbm.at[:, pl.ds(0, CH)], eidx[u], isem[u]).wait()

    idx_load(0, 0)
    idx_load(1, 1)
    wait_idx(0)
    if compute is not None:
        compute(0, 0)
    issue_gather(0, 0)

    def step(t, carry):
        for u in range(4):
            r, r1, u1, u2 = u % 2, (u + 1) % 2, (u + 1) % 4, (u + 2) % 4
            wait_gather(r)
            issue_scatter(u, r)
            if u < 3:
                wait_idx(u1)
                if u == 0:
                    @pl.when(t >= 1)
                    def _():
                        wait_scatter(r1)
                else:
                    wait_scatter(r1)
                if compute is not None:
                    compute(u1, r1)
                issue_gather(u1, r1)
            else:
                @pl.when(t < T - 1)
                def _():
                    wait_idx(u1)
                    wait_scatter(r1)
                    if compute is not None:
                        compute(u1, r1)
                    issue_gather(u1, r1)
            if u < 2:
                idx_load(4 * t + u + 2, u2)
            else:
                @pl.when(t < T - 1)
                def _():
                    idx_load(4 * t + u + 2, u2)
        return carry

    lax.fori_loop(0, T, step, 0)
    wait_scatter(0)
    wait_scatter(1)


def _segsum_body(edges_hbm, p_hbm, z_hbm, agg_out,
                 e0v, e1v, e2v, e3v, rows0, rows1, acc_sh,
                 is0, is1, is2, is3, gs0, gs1, ss0, ss1):
    c = lax.axis_index("c")
    s = lax.axis_index("s")

    zrows = NTR // NS
    pltpu.sync_copy(z_hbm.at[pl.ds(s * zrows, zrows)],
                    acc_sh.at[pl.ds(s * zrows, zrows)])
    plsc.subcore_barrier()

    base = (c * NS + s) * EB_TILE
    eidx = [e0v, e1v, e2v, e3v]
    rows = [rows0, rows1]
    isem = [is0, is1, is2, is3]
    gsem = [gs0, gs1]
    ssem = [ss0, ss1]

    def issue_gather(u, r):
        pltpu.async_copy(p_hbm.at[eidx[u].at[0]], rows[r], gsem[r])

    def wait_gather(r):
        pltpu.make_async_copy(p_hbm.at[eidx[0].at[0]], rows[r],
                              gsem[r]).wait()

    def issue_scatter(u, r):
        pltpu.async_copy(rows[r], acc_sh.at[eidx[u].at[1]], ssem[r],
                         add=True)

    def wait_scatter(r):
        pltpu.make_async_copy(rows[r], acc_sh.at[eidx[0].at[1]],
                              ssem[r]).wait()

    _edge_sweep(EB_CHUNKS, base, edges_hbm, eidx, isem,
                issue_gather, wait_gather, issue_scatter, wait_scatter)

    plsc.subcore_barrier()
    pltpu.sync_copy(acc_sh.at[pl.ds(s * zrows, zrows)],
                    agg_out.at[c, pl.ds(s * zrows, zrows)])


def _segsum(edges, p, z):
    mesh = plsc.VectorSubcoreMesh(core_axis_name="c", subcore_axis_name="s",
                                  num_cores=NC, num_subcores=NS)
    return pl.kernel(
        _segsum_body,
        out_type=jax.ShapeDtypeStruct((NC, NTR, 128), jnp.float32),
        mesh=mesh,
        scratch_types=[
            pltpu.VMEM((2, CH), jnp.int32),
            pltpu.VMEM((2, CH), jnp.int32),
            pltpu.VMEM((2, CH), jnp.int32),
            pltpu.VMEM((2, CH), jnp.int32),
            pltpu.VMEM((CH, 128), jnp.float32),
            pltpu.VMEM((CH, 128), jnp.float32),
            pltpu.VMEM_SHARED((NTR, 128), jnp.float32),
            pltpu.SemaphoreType.DMA,
            pltpu.SemaphoreType.DMA,
            pltpu.SemaphoreType.DMA,
            pltpu.SemaphoreType.DMA,
            pltpu.SemaphoreType.DMA,
            pltpu.SemaphoreType.DMA,
            pltpu.SemaphoreType.DMA,
            pltpu.SemaphoreType.DMA,
        ],
        compiler_params=pltpu.CompilerParams(use_tc_tiling_on_sc=True),
    )(edges, p, z)


def _softmax_body(h_ref, w_ref, agg_ref, s_ref, t_ref, reg_ref):
    i = pl.program_id(0)
    n = pl.num_programs(0)
    blk = h_ref.shape[0]
    q = jnp.dot(h_ref[...], w_ref[...], preferred_element_type=jnp.float32)
    aggs = agg_ref[0, :, 0:K] + agg_ref[1, :, 0:K]
    deg = agg_ref[0, :, K:K + 1] + agg_ref[1, :, K:K + 1]
    logits = q + aggs / jnp.maximum(deg, 1.0)
    m = jnp.max(logits, axis=1, keepdims=True)
    ex = jnp.exp(logits - m)
    sm = ex / jnp.sum(ex, axis=1, keepdims=True)
    s_ref[...] = sm
    t_ref[...] = jnp.concatenate(
        [sm, jnp.zeros((blk, 128), jnp.float32), sm], axis=1)
    part = jnp.sum(sm * sm - sm * jnp.log(sm + 1e-12)).reshape(1, 1)
    prev = jnp.where(i == 0, jnp.zeros((1, 1), jnp.float32), reg_ref[...])
    tot = prev + part
    reg_ref[...] = jnp.where(i == n - 1, tot / N, tot)


def _softmax(h, w_self, agg):
    blk = 1000
    return pl.pallas_call(
        _softmax_body,
        grid=(N // blk,),
        in_specs=[
            pl.BlockSpec((blk, D), lambda i: (i, 0)),
            pl.BlockSpec((D, K), lambda i: (0, 0)),
            pl.BlockSpec((NC, blk, 128), lambda i: (0, i, 0)),
        ],
        out_specs=[
            pl.BlockSpec((blk, K), lambda i: (i, 0)),
            pl.BlockSpec((blk, 4 * K), lambda i: (i, 0)),
            pl.BlockSpec((1, 1), lambda i: (0, 0)),
        ],
        out_shape=[
            jax.ShapeDtypeStruct((N, K), jnp.float32),
            jax.ShapeDtypeStruct((N, 4 * K), jnp.float32),
            jax.ShapeDtypeStruct((1, 1), jnp.float32),
        ],
    )(h, w_self, agg)


def _part_body(edges_hbm, part_out, cnt_out,
               eidx, seg_src, seg_dst, cntv, ptr_sm):
    c = lax.axis_index("c")
    s = lax.axis_index("s")
    w = c * NS + s
    base = w * EB_TILE

    ptr_sm[0] = 0
    ptr_sm[1] = EB_TILE

    def chunk(i, carry):
        pltpu.sync_copy(edges_hbm.at[:, pl.ds(base + i * CH, CH)], eidx)
        for j in range(CH // 16):
            sl = pl.ds(j * 16, 16)
            sv = eidx[0, sl]
            dv = eidx[1, sl]
            mi = jnp.where(sv < N // 2, 1, 0)
            cf = plsc.cumsum(mi)
            cb = plsc.cumsum(1 - mi)
            fp = ptr_sm[0]
            bp = ptr_sm[1]
            pos = jnp.where(mi == 1, fp + cf - 1, bp - cb)
            plsc.store_scatter(seg_src, [pos], sv)
            plsc.store_scatter(seg_dst, [pos], dv)
            nf = jnp.sum(mi)
            ptr_sm[0] = fp + nf
            ptr_sm[1] = bp - (16 - nf)
        return carry

    lax.fori_loop(0, EB_CHUNKS, chunk, 0)

    pltpu.sync_copy(seg_src, part_out.at[0, pl.ds(base, EB_TILE)])
    pltpu.sync_copy(seg_dst, part_out.at[1, pl.ds(base, EB_TILE)])
    cntv[:] = lax.iota(jnp.int32, 16) * 0 + ptr_sm[0]
    pltpu.sync_copy(cntv, cnt_out.at[w])


def _partition(edges):
    mesh = plsc.VectorSubcoreMesh(core_axis_name="c", subcore_axis_name="s",
                                  num_cores=NC, num_subcores=NS)
    return pl.kernel(
        _part_body,
        out_type=(
            jax.ShapeDtypeStruct((2, EPAD), jnp.int32),
            jax.ShapeDtypeStruct((NW, 16), jnp.int32),
        ),
        mesh=mesh,
        scratch_types=[
            pltpu.VMEM((2, CH), jnp.int32),
            pltpu.VMEM((EB_TILE,), jnp.int32),
            pltpu.VMEM((EB_TILE,), jnp.int32),
            pltpu.VMEM((16,), jnp.int32),
            pltpu.SMEM((2,), jnp.int32),
        ],
        compiler_params=pltpu.CompilerParams(use_tc_tiling_on_sc=True,
                                             needs_layout_passes=False),
    )(edges)


def _as_body(edges_hbm, cnt_hbm, t2_hbm, z_hbm, as_out,
             eidx, gidx, lidx, rows, cntv, acc_sh, sem):
    c = lax.axis_index("c")
    s = lax.axis_index("s")

    zrows = NTR // NS
    pltpu.sync_copy(z_hbm.at[pl.ds(s * zrows, zrows)],
                    acc_sh.at[pl.ds(s * zrows, zrows)])
    plsc.subcore_barrier()

    lo = 2 * c * NPER
    mid = lo + NPER
    hi = lo + 2 * NPER

    def do_chunk(e0, carry):
        pltpu.sync_copy(edges_hbm.at[:, pl.ds(e0, CH)], eidx)
        for j in range(CH // 16):
            sl = pl.ds(j * 16, 16)
            sv = eidx[0, sl]
            dv = eidx[1, sl]
            gidx[sl] = 2 * sv + (dv & 1)
            valid = (sv >= lo) & (sv < hi) & (dv < N)
            half_off = jnp.where(sv >= mid, N // 2, 0)
            lidx[sl] = jnp.where(valid, half_off + (dv >> 1),
                                 TRASH + (dv & 63))
        pltpu.async_copy(t2_hbm.at[gidx], rows, sem).wait()
        pltpu.sync_copy(rows, acc_sh.at[lidx], add=True)
        return carry

    for q in range(2):
        t = 2 * s + q
        pltpu.sync_copy(cnt_hbm.at[t], cntv)
        cnt = cntv[...][0]
        full = cnt >> 7
        start = jnp.where(c == 0, 0, full)
        nch = jnp.where(c == 0, (cnt + CH - 1) >> 7, EB_CHUNKS - full)
        seg = t * EB_TILE

        def body(i, carry):
            return do_chunk(seg + (start + i) * CH, carry)

        lax.fori_loop(0, nch, body, 0)

    plsc.subcore_barrier()
    pltpu.sync_copy(acc_sh.at[pl.ds(s * zrows, zrows)],
                    as_out.at[c, pl.ds(s * zrows, zrows)])


def _as_scatter(edges, cnts, t2, z):
    mesh = plsc.VectorSubcoreMesh(core_axis_name="c", subcore_axis_name="s",
                                  num_cores=NC, num_subcores=NS)
    return pl.kernel(
        _as_body,
        out_type=jax.ShapeDtypeStruct((NC, NTR, 128), jnp.float32),
        mesh=mesh,
        scratch_types=[
            pltpu.VMEM((2, CH), jnp.int32),
            pltpu.VMEM((CH,), jnp.int32),
            pltpu.VMEM((CH,), jnp.int32),
            pltpu.VMEM((CH, 128), jnp.float32),
            pltpu.VMEM((16,), jnp.int32),
            pltpu.VMEM_SHARED((NTR, 128), jnp.float32),
            pltpu.SemaphoreType.DMA,
        ],
        compiler_params=pltpu.CompilerParams(use_tc_tiling_on_sc=True,
                                             needs_layout_passes=False),
    )(edges, cnts, t2, z)


def _adj_body(s_ref, as_ref, o_ref):
    sg = s_ref[0]
    blocks = []
    for gs in range(B):
        a = as_ref[gs, 0]
        blocks.append(lax.dot_general(
            sg, a, (((0,), (0,)), ((), ())),
            preferred_element_type=jnp.float32))
    o_ref[...] = jnp.concatenate(blocks, axis=1)


def _adj(s_r, as_r):
    return pl.pallas_call(
        _adj_body,
        grid=(B,),
        in_specs=[
            pl.BlockSpec((1, NPER, K), lambda gd: (gd, 0, 0)),
            pl.BlockSpec((B, 1, NPER, K), lambda gd: (0, gd, 0, 0)),
        ],
        out_specs=pl.BlockSpec((K, B * K), lambda gd: (gd, 0)),
        out_shape=jax.ShapeDtypeStruct((B * K, B * K), jnp.float32),
    )(s_r, as_r)


@jax.jit
def kernel(h, edge_index, W_self, W_neigh):
    npad = EPAD - E
    pad = jnp.concatenate(
        [jnp.zeros((1, npad), jnp.int32),
         jnp.full((1, npad), N, jnp.int32)], axis=0)
    edges = jnp.concatenate([edge_index, pad], axis=1)
    z = jnp.zeros((NTR, 128), jnp.float32)

    p128 = _project(h, W_neigh)
    agg = _segsum(edges, p128, z)
    part, cnts = _partition(edges)
    s_mat, t_mat, reg = _softmax(h, W_self, agg)
    t2 = t_mat.reshape(2 * N, 128)
    as4 = _as_scatter(part, cnts, t2, z)
    flat = as4.reshape(NC, 2 * NTR, K)
    s_r = s_mat.reshape(B, NPER, K)
    as_r = flat[:, :2 * N].reshape(NC, 2, B, NPER, K).reshape(B, B, NPER, K)
    adj_new = _adj(s_r, as_r)
    return adj_new, reg[0, 0]
